# baseline (speedup 1.0000x reference)
import functools

import jax
import jax.numpy as jnp
from jax import lax
from jax.experimental import pallas as pl
from jax.experimental.pallas import tpu as pltpu
from jax.experimental.pallas import tpu_sc as plsc

_NC = 2
_NS = 16
_NW = _NC * _NS

_TOTAL = 4 * 4096 * 2048
_PER_W = _TOTAL // _NW
_CH = 16384
_NCH = _PER_W // _CH
_NBUF = 4
_K = _NBUF // 2
_NG = _NCH // _NBUF

_mesh = plsc.VectorSubcoreMesh(core_axis_name="c", subcore_axis_name="s")


@functools.partial(
    pl.kernel,
    mesh=_mesh,
    out_type=jax.ShapeDtypeStruct((_TOTAL,), jnp.float32),
    scratch_types=(
        [pltpu.VMEM_SHARED((_NS, _NBUF, _CH), jnp.float32)]
        + [pltpu.SemaphoreType.DMA] * (2 * _NBUF)
    ),
)
def _sc_copy(x_hbm, o_hbm, buf, *sems):
    isems = sems[:_NBUF]
    osems = sems[_NBUF:]
    sid = lax.axis_index("s")
    wid = sid * _NC + lax.axis_index("c")
    base = wid * _PER_W

    def in_cp(idx, b):
        return pltpu.make_async_copy(
            x_hbm.at[pl.ds(base + idx * _CH, _CH)], buf.at[sid, b], isems[b])

    def out_cp(idx, b):
        return pltpu.make_async_copy(
            buf.at[sid, b], o_hbm.at[pl.ds(base + idx * _CH, _CH)], osems[b])

    for b in range(_K):
        in_cp(b, b).start()

    def group(g, carry):
        i0 = g * _NBUF
        for b in range(_NBUF):
            idx = i0 + b
            in_cp(idx, b).wait()
            out_cp(idx, b).start()

            ob = (b + _K) % _NBUF

            @pl.when(idx >= _K)
            def _():
                out_cp(idx - _K, ob).wait()

            @pl.when(idx + _K < _NCH)
            def _():
                in_cp(idx + _K, ob).start()

        return carry

    lax.fori_loop(0, _NG, group, 0)
    for t in range(_K):
        idx = _NCH - _K + t
        out_cp(idx, idx % _NBUF).wait()


def kernel(x):
    b, s, d = x.shape
    out = _sc_copy(x.reshape(-1))
    return out.reshape(b, s, d)


# speedup vs baseline: 3.9586x; 3.9586x over previous
import jax
import jax.numpy as jnp
from jax.experimental import pallas as pl
from jax.experimental.pallas import tpu as pltpu

_BLOCK_ROWS = 512


def _copy_block(x_ref, o_ref):
    o_ref[...] = x_ref[...]


def kernel(x):
    b, s, d = x.shape
    x2 = x.reshape(b * s, d)
    out = pl.pallas_call(
        _copy_block,
        grid=(x2.shape[0] // _BLOCK_ROWS,),
        in_specs=[pl.BlockSpec((_BLOCK_ROWS, d), lambda i: (i, 0))],
        out_specs=pl.BlockSpec((_BLOCK_ROWS, d), lambda i: (i, 0)),
        out_shape=jax.ShapeDtypeStruct(x2.shape, x2.dtype),
    )(x2)
    return out.reshape(b, s, d)


# speedup vs baseline: 4.0401x; 1.0206x over previous
import jax
import jax.numpy as jnp
from jax.experimental import pallas as pl
from jax.experimental.pallas import tpu as pltpu

_BLOCK_ROWS = 1024


def _copy_block(x_ref, o_ref):
    o_ref[...] = x_ref[...]


def kernel(x):
    b, s, d = x.shape
    x2 = x.reshape(b * s, d)
    out = pl.pallas_call(
        _copy_block,
        grid=(x2.shape[0] // _BLOCK_ROWS,),
        in_specs=[pl.BlockSpec((_BLOCK_ROWS, d), lambda i: (i, 0))],
        out_specs=pl.BlockSpec((_BLOCK_ROWS, d), lambda i: (i, 0)),
        out_shape=jax.ShapeDtypeStruct(x2.shape, x2.dtype),
    )(x2)
    return out.reshape(b, s, d)
